# baseline (speedup 1.0000x reference)
import functools

import jax
import jax.numpy as jnp
from jax import lax
from jax.experimental import pallas as pl
from jax.experimental.pallas import tpu as pltpu
from jax.experimental.pallas import tpu_sc as plsc

D_MODEL = 1024
T = 8192

_NC = 2
_NS = 16
_NW = _NC * _NS
_BPW = T // _NW
_C = 32
_NCHUNK = _BPW // _C


@functools.partial(
    pl.kernel,
    mesh=plsc.VectorSubcoreMesh(core_axis_name="c", subcore_axis_name="s"),
    out_type=jax.ShapeDtypeStruct((1, T, D_MODEL), jnp.float32),
    scratch_types=[
        pltpu.VMEM((_BPW,), jnp.int32),
        pltpu.VMEM((_C, D_MODEL), jnp.float32),
        pltpu.VMEM((_C, D_MODEL), jnp.float32),
        pltpu.SemaphoreType.DMA,
        pltpu.SemaphoreType.DMA,
        pltpu.SemaphoreType.DMA,
        pltpu.SemaphoreType.DMA,
    ],
)
def _gather_rows(types_hbm, emb_hbm, out_hbm, idx_v, buf0, buf1,
                 g0, g1, w0, w1):
    wid = lax.axis_index("s") * _NC + lax.axis_index("c")
    base = wid * _BPW
    bufs = (buf0, buf1)
    gsem = (g0, g1)
    wsem = (w0, w1)
    pltpu.sync_copy(types_hbm.at[pl.ds(base, _C)], idx_v.at[pl.ds(0, _C)])
    pltpu.async_copy(emb_hbm.at[idx_v.at[pl.ds(0, _C)]], bufs[0], gsem[0])
    pltpu.sync_copy(types_hbm.at[pl.ds(base + _C, _BPW - _C)],
                    idx_v.at[pl.ds(_C, _BPW - _C)])
    pltpu.async_copy(emb_hbm.at[idx_v.at[pl.ds(_C, _C)]], bufs[1], gsem[1])

    def _wait_gather(b):
        pltpu.make_async_copy(emb_hbm.at[pl.ds(0, _C)], bufs[b],
                              gsem[b]).wait()

    def _wait_write(b):
        pltpu.make_async_copy(bufs[b], out_hbm.at[0, pl.ds(0, _C)],
                              wsem[b]).wait()

    def _pair(g, carry):
        off = base + 2 * g * _C

        @pl.when(g > 0)
        def _():
            for b in (0, 1):
                _wait_write(b)
                pltpu.async_copy(
                    emb_hbm.at[idx_v.at[pl.ds(2 * g * _C + b * _C, _C)]],
                    bufs[b], gsem[b])

        for b in (0, 1):
            _wait_gather(b)
            pltpu.async_copy(
                bufs[b], out_hbm.at[0, pl.ds(off + b * _C, _C)], wsem[b])
        return carry

    lax.fori_loop(0, _NCHUNK // 2, _pair, 0)
    _wait_write(0)
    _wait_write(1)


def kernel(types, emb):
    return _gather_rows(types.astype(jnp.int32), emb)


# speedup vs baseline: 1.0114x; 1.0114x over previous
import functools

import jax
import jax.numpy as jnp
from jax import lax
from jax.experimental import pallas as pl
from jax.experimental.pallas import tpu as pltpu
from jax.experimental.pallas import tpu_sc as plsc

D_MODEL = 1024
T = 8192

_NC = 2
_NS = 16
_NW = _NC * _NS
_BPW = T // _NW
_C = 16
_NCHUNK = _BPW // _C


@functools.partial(
    pl.kernel,
    mesh=plsc.VectorSubcoreMesh(core_axis_name="c", subcore_axis_name="s"),
    out_type=jax.ShapeDtypeStruct((1, T, D_MODEL), jnp.float32),
    scratch_types=[
        pltpu.VMEM((_BPW,), jnp.int32),
        pltpu.VMEM((_C, D_MODEL), jnp.float32),
        pltpu.VMEM((_C, D_MODEL), jnp.float32),
        pltpu.SemaphoreType.DMA,
        pltpu.SemaphoreType.DMA,
        pltpu.SemaphoreType.DMA,
        pltpu.SemaphoreType.DMA,
    ],
)
def _gather_rows(types_hbm, emb_hbm, out_hbm, idx_v, buf0, buf1,
                 g0, g1, w0, w1):
    wid = lax.axis_index("s") * _NC + lax.axis_index("c")
    base = wid * _BPW
    bufs = (buf0, buf1)
    gsem = (g0, g1)
    wsem = (w0, w1)
    pltpu.sync_copy(types_hbm.at[pl.ds(base, _C)], idx_v.at[pl.ds(0, _C)])
    gh = [None] * _NCHUNK
    wh = [None] * _NCHUNK
    gh[0] = pltpu.async_copy(
        emb_hbm.at[idx_v.at[pl.ds(0, _C)]], bufs[0], gsem[0])
    pltpu.sync_copy(types_hbm.at[pl.ds(base + _C, _BPW - _C)],
                    idx_v.at[pl.ds(_C, _BPW - _C)])
    for c in range(_NCHUNK):
        b = c % 2
        if c + 1 < _NCHUNK:
            nb = (c + 1) % 2
            if c >= 1:
                wh[c - 1].wait()
            gh[c + 1] = pltpu.async_copy(
                emb_hbm.at[idx_v.at[pl.ds((c + 1) * _C, _C)]],
                bufs[nb], gsem[nb])
        gh[c].wait()
        wh[c] = pltpu.async_copy(
            bufs[b], out_hbm.at[0, pl.ds(base + c * _C, _C)], wsem[b])
    wh[_NCHUNK - 2].wait()
    wh[_NCHUNK - 1].wait()


def kernel(types, emb):
    return _gather_rows(types.astype(jnp.int32), emb)


# speedup vs baseline: 1.0155x; 1.0040x over previous
import functools

import jax
import jax.numpy as jnp
from jax import lax
from jax.experimental import pallas as pl
from jax.experimental.pallas import tpu as pltpu
from jax.experimental.pallas import tpu_sc as plsc

D_MODEL = 1024
T = 8192

_NC = 2
_NS = 16
_NW = _NC * _NS
_BPW = T // _NW
_C = 32
_NCHUNK = _BPW // _C


@functools.partial(
    pl.kernel,
    mesh=plsc.VectorSubcoreMesh(core_axis_name="c", subcore_axis_name="s"),
    out_type=jax.ShapeDtypeStruct((1, T, D_MODEL), jnp.float32),
    scratch_types=[
        pltpu.VMEM((_BPW,), jnp.int32),
        pltpu.VMEM((_C, D_MODEL), jnp.float32),
        pltpu.VMEM((_C, D_MODEL), jnp.float32),
        pltpu.SemaphoreType.DMA,
        pltpu.SemaphoreType.DMA,
        pltpu.SemaphoreType.DMA,
        pltpu.SemaphoreType.DMA,
    ],
)
def _gather_rows(types_hbm, emb_hbm, out_hbm, idx_v, buf0, buf1,
                 g0, g1, w0, w1):
    wid = lax.axis_index("s") * _NC + lax.axis_index("c")
    base = wid * _BPW
    bufs = (buf0, buf1)
    gsem = (g0, g1)
    wsem = (w0, w1)
    pltpu.sync_copy(types_hbm.at[pl.ds(base, _C)], idx_v.at[pl.ds(0, _C)])
    gh = [None] * _NCHUNK
    wh = [None] * _NCHUNK
    gh[0] = pltpu.async_copy(
        emb_hbm.at[idx_v.at[pl.ds(0, _C)]], bufs[0], gsem[0])
    pltpu.sync_copy(types_hbm.at[pl.ds(base + _C, _BPW - _C)],
                    idx_v.at[pl.ds(_C, _BPW - _C)])
    for c in range(_NCHUNK):
        b = c % 2
        if c + 1 < _NCHUNK:
            nb = (c + 1) % 2
            if c >= 1:
                wh[c - 1].wait()
            gh[c + 1] = pltpu.async_copy(
                emb_hbm.at[idx_v.at[pl.ds((c + 1) * _C, _C)]],
                bufs[nb], gsem[nb])
        gh[c].wait()
        wh[c] = pltpu.async_copy(
            bufs[b], out_hbm.at[0, pl.ds(base + c * _C, _C)], wsem[b])
    wh[_NCHUNK - 2].wait()
    wh[_NCHUNK - 1].wait()


def kernel(types, emb):
    return _gather_rows(types.astype(jnp.int32), emb)


# speedup vs baseline: 1.0411x; 1.0252x over previous
import functools

import jax
import jax.numpy as jnp
from jax import lax
from jax.experimental import pallas as pl
from jax.experimental.pallas import tpu as pltpu
from jax.experimental.pallas import tpu_sc as plsc

D_MODEL = 1024
T = 8192

_NC = 2
_NS = 16
_NW = _NC * _NS
_BPW = T // _NW
_C = 32
_NCHUNK = _BPW // _C
_NBUF = 3


@functools.partial(
    pl.kernel,
    mesh=plsc.VectorSubcoreMesh(core_axis_name="c", subcore_axis_name="s"),
    out_type=jax.ShapeDtypeStruct((1, T, D_MODEL), jnp.float32),
    scratch_types=[
        pltpu.VMEM((_BPW,), jnp.int32),
        pltpu.VMEM((_C, D_MODEL), jnp.float32),
        pltpu.VMEM((_C, D_MODEL), jnp.float32),
        pltpu.VMEM((_C, D_MODEL), jnp.float32),
        pltpu.SemaphoreType.DMA,
        pltpu.SemaphoreType.DMA,
        pltpu.SemaphoreType.DMA,
        pltpu.SemaphoreType.DMA,
        pltpu.SemaphoreType.DMA,
        pltpu.SemaphoreType.DMA,
    ],
)
def _gather_rows(types_hbm, emb_hbm, out_hbm, idx_v, buf0, buf1, buf2,
                 g0, g1, g2, w0, w1, w2):
    wid = lax.axis_index("s") * _NC + lax.axis_index("c")
    base = wid * _BPW
    bufs = (buf0, buf1, buf2)
    gsem = (g0, g1, g2)
    wsem = (w0, w1, w2)

    def gather(c):
        return pltpu.async_copy(
            emb_hbm.at[idx_v.at[pl.ds(c * _C, _C)]],
            bufs[c % _NBUF], gsem[c % _NBUF])

    pltpu.sync_copy(types_hbm.at[pl.ds(base, _C)], idx_v.at[pl.ds(0, _C)])
    gh = [None] * _NCHUNK
    wh = [None] * _NCHUNK
    gh[0] = gather(0)
    pltpu.sync_copy(types_hbm.at[pl.ds(base + _C, _BPW - _C)],
                    idx_v.at[pl.ds(_C, _BPW - _C)])
    gh[1] = gather(1)
    for c in range(_NCHUNK):
        if c + 2 < _NCHUNK:
            if c >= 1:
                wh[c - 1].wait()
            gh[c + 2] = gather(c + 2)
        gh[c].wait()
        wh[c] = pltpu.async_copy(
            bufs[c % _NBUF], out_hbm.at[0, pl.ds(base + c * _C, _C)],
            wsem[c % _NBUF])
    wh[_NCHUNK - 3].wait()
    wh[_NCHUNK - 2].wait()
    wh[_NCHUNK - 1].wait()


def kernel(types, emb):
    return _gather_rows(types.astype(jnp.int32), emb)
